# baseline (speedup 1.0000x reference)
import functools

import jax
import jax.numpy as jnp
from jax import lax
from jax.experimental import pallas as pl
from jax.experimental.pallas import tpu as pltpu
from jax.experimental.pallas import tpu_sc as plsc

L = 16
NROWS = 8192
NCOLS = 4096
NWORK = 32
ROWS_PER_W = NROWS // NWORK
R = 4
NCHUNK = ROWS_PER_W // R
NSUPER = NCHUNK // 2
CHUNK_BYTES = R * NCOLS * 4
CVEC = NCOLS // L
U = 8
NITER = CVEC // U
NM = 6
BITS = 24

_mesh = plsc.VectorSubcoreMesh(core_axis_name="c", subcore_axis_name="s")

_GDN = lax.GatherDimensionNumbers(
    offset_dims=(), collapsed_slice_dims=(0,), start_index_map=(0,)
)


def _shuf(v, idx):
    return lax.gather(
        v,
        idx[:, None],
        dimension_numbers=_GDN,
        slice_sizes=(1,),
        mode=lax.GatherScatterMode.PROMISE_IN_BOUNDS,
    )


def _xlane(v, op):
    r = v
    for k in (1, 2, 4, 8):
        idx = lax.iota(jnp.int32, L) ^ k
        r = op(r, _shuf(r, idx))
    return r


def _prefix(x):
    lane = lax.iota(jnp.int32, L)
    r = x
    for k in (1, 2, 4, 8):
        sh = _shuf(r, jnp.maximum(lane - k, 0))
        r = r + jnp.where(lane >= k, sh, 0.0)
    return r


@functools.partial(
    pl.kernel,
    mesh=_mesh,
    out_type=jax.ShapeDtypeStruct((NROWS, NCOLS), jnp.float32),
    scratch_types=[
        pltpu.VMEM((R, NCOLS), jnp.float32),
        pltpu.VMEM((R, NCOLS), jnp.float32),
        pltpu.VMEM((R, NCOLS), jnp.float32),
        pltpu.VMEM((R, NCOLS), jnp.float32),
        pltpu.VMEM((NCOLS + 4 * L,), jnp.float32),
        pltpu.VMEM((L,), jnp.float32),
        pltpu.SemaphoreType.DMA,
        pltpu.SemaphoreType.DMA,
        pltpu.SemaphoreType.DMA,
        pltpu.SemaphoreType.DMA,
    ],
)
def _sparsemax_sc(x_hbm, out_hbm, in0, in1, out0, out1, cbuf, taubuf,
                  si0, si1, so0, so1):
    wid = lax.axis_index("s") * 2 + lax.axis_index("c")
    base_row = wid * ROWS_PER_W

    def process_chunk(in_v, out_v):
        def row_body(r, _):
            row = in_v.at[r]

            def max_body(i, ms):
                return tuple(
                    jnp.maximum(ms[j], row[pl.ds(i * (U * L) + j * L, L)])
                    for j in range(U)
                )

            ms = lax.fori_loop(
                0, NITER, max_body,
                tuple(jnp.full((L,), -jnp.inf, jnp.float32) for _ in range(U)),
            )
            m = ms[0]
            for j in range(1, U):
                m = jnp.maximum(m, ms[j])
            top = _xlane(m, jnp.maximum)
            thr = top - 1.0

            def app_body(i, vofs):
                vs = [row[pl.ds(i * (U * L) + j * L, L)] for j in range(U)]
                am = jnp.zeros((L,), jnp.float32)
                bm = jnp.zeros((L,), jnp.float32)
                for j in range(4):
                    am = am + jnp.where(vs[j] > thr, float(1 << (5 * j)), 0.0)
                    bm = bm + jnp.where(vs[4 + j] > thr, float(1 << (5 * j)), 0.0)
                a0 = _xlane(am, jnp.add)[0].astype(jnp.int32)
                b0 = _xlane(bm, jnp.add)[0].astype(jnp.int32)
                for j in range(U):
                    cbuf[pl.ds(vofs, L)] = vs[j]
                    w = a0 if j < 4 else b0
                    d = lax.shift_right_logical(w, 5 * (j % 4)) & 31
                    vofs = vofs + lax.shift_left(jnp.minimum(d, 1), 4)
                return vofs

            vofs = lax.fori_loop(0, NITER, app_body, jnp.int32(0))

            for j in range(4):
                cbuf[pl.ds(vofs + j * L, L)] = thr
            nquad = lax.shift_right_logical(vofs + 4 * L, 6)

            def scan_sk(tau_c):
                def scan_body(p, accs):
                    accs = list(accs)
                    q = p * (4 * L)
                    for d in range(4):
                        v = cbuf[pl.ds(q + d * L, L)]
                        sel = v > tau_c
                        accs[d] = accs[d] + jnp.where(sel, v, 0.0)
                        accs[4 + d] = accs[4 + d] + jnp.where(sel, 1.0, 0.0)
                    return tuple(accs)

                z = jnp.zeros((L,), jnp.float32)
                r = lax.fori_loop(0, nquad, scan_body, (z,) * 8)
                s_v = (r[0] + r[1]) + (r[2] + r[3])
                k_v = (r[4] + r[5]) + (r[6] + r[7])
                return _xlane(s_v, jnp.add), _xlane(k_v, jnp.add)

            def mic_body(it, carry):
                tau_c, _ = carry
                s_t, k_t = scan_sk(tau_c)
                return ((s_t - 1.0) / k_t, tau_c)

            tau_m, tau_p = lax.fori_loop(0, NM, mic_body, (thr, thr - 1.0))
            taubuf[...] = tau_m

            @pl.when(tau_m[0] != tau_p[0])
            def _():
                def bis_body(b, carry):
                    lo, hi = carry
                    mid = 0.5 * (lo + hi)
                    s_t, k_t = scan_sk(mid)
                    big = (s_t - k_t * mid) > 1.0
                    return jnp.where(big, mid, lo), jnp.where(big, hi, mid)

                lo, hi = lax.fori_loop(0, BITS, bis_body, (tau_m, top))
                taubuf[...] = 0.5 * (lo + hi)

            tau = taubuf[...]

            orow = out_v.at[r]

            def out_body(i, _):
                for j in range(U):
                    off = i * (U * L) + j * L
                    orow[pl.ds(off, L)] = jnp.maximum(row[pl.ds(off, L)] - tau, 0.0)
                return 0

            lax.fori_loop(0, NITER, out_body, 0)
            return 0

        lax.fori_loop(0, R, row_body, 0)

    pltpu.async_copy(x_hbm.at[pl.ds(base_row, R)], in0, si0)

    def super_body(t, _):
        rowa = base_row + (2 * t) * R
        rowb = rowa + R

        pltpu.make_async_copy(x_hbm.at[pl.ds(rowa, R)], in0, si0).wait()
        pltpu.async_copy(x_hbm.at[pl.ds(rowb, R)], in1, si1)

        @pl.when(t > 0)
        def _():
            pltpu.make_async_copy(x_hbm.at[pl.ds(rowa, R)], out0, so0).wait()

        process_chunk(in0, out0)
        pltpu.async_copy(out0, out_hbm.at[pl.ds(rowa, R)], so0)

        pltpu.make_async_copy(x_hbm.at[pl.ds(rowb, R)], in1, si1).wait()

        @pl.when(t < NSUPER - 1)
        def _():
            pltpu.async_copy(x_hbm.at[pl.ds(rowb + R, R)], in0, si0)

        @pl.when(t > 0)
        def _():
            pltpu.make_async_copy(x_hbm.at[pl.ds(rowb, R)], out1, so1).wait()

        process_chunk(in1, out1)
        pltpu.async_copy(out1, out_hbm.at[pl.ds(rowb, R)], so1)
        return 0

    lax.fori_loop(0, NSUPER, super_body, 0)

    pltpu.make_async_copy(x_hbm.at[pl.ds(base_row, R)], out0, so0).wait()
    pltpu.make_async_copy(x_hbm.at[pl.ds(base_row, R)], out1, so1).wait()


def kernel(x):
    out = _sparsemax_sc(x.reshape(NROWS, NCOLS))
    return out.reshape(x.shape)


# speedup vs baseline: 1.0567x; 1.0567x over previous
import functools

import jax
import jax.numpy as jnp
from jax import lax
from jax.experimental import pallas as pl
from jax.experimental.pallas import tpu as pltpu
from jax.experimental.pallas import tpu_sc as plsc

L = 16
NROWS = 8192
NCOLS = 4096
NWORK = 32
ROWS_PER_W = NROWS // NWORK
R = 4
NCHUNK = ROWS_PER_W // R
NSUPER = NCHUNK // 2
CHUNK_BYTES = R * NCOLS * 4
CVEC = NCOLS // L
U = 8
NITER = CVEC // U
NM = 6
BITS = 24

_mesh = plsc.VectorSubcoreMesh(core_axis_name="c", subcore_axis_name="s")

_GDN = lax.GatherDimensionNumbers(
    offset_dims=(), collapsed_slice_dims=(0,), start_index_map=(0,)
)


def _shuf(v, idx):
    return lax.gather(
        v,
        idx[:, None],
        dimension_numbers=_GDN,
        slice_sizes=(1,),
        mode=lax.GatherScatterMode.PROMISE_IN_BOUNDS,
    )


def _xlane(v, op):
    r = v
    for k in (1, 2, 4, 8):
        idx = lax.iota(jnp.int32, L) ^ k
        r = op(r, _shuf(r, idx))
    return r


def _prefix(x):
    lane = lax.iota(jnp.int32, L)
    r = x
    for k in (1, 2, 4, 8):
        sh = _shuf(r, jnp.maximum(lane - k, 0))
        r = r + jnp.where(lane >= k, sh, 0.0)
    return r


@functools.partial(
    pl.kernel,
    mesh=_mesh,
    out_type=jax.ShapeDtypeStruct((NROWS, NCOLS), jnp.float32),
    scratch_types=[
        pltpu.VMEM((R, NCOLS), jnp.float32),
        pltpu.VMEM((R, NCOLS), jnp.float32),
        pltpu.VMEM((R, NCOLS), jnp.float32),
        pltpu.VMEM((R, NCOLS), jnp.float32),
        pltpu.VMEM((NCOLS + 4 * L,), jnp.float32),
        pltpu.VMEM((L,), jnp.float32),
        pltpu.SemaphoreType.DMA,
        pltpu.SemaphoreType.DMA,
        pltpu.SemaphoreType.DMA,
        pltpu.SemaphoreType.DMA,
    ],
)
def _sparsemax_sc(x_hbm, out_hbm, in0, in1, out0, out1, cbuf, taubuf,
                  si0, si1, so0, so1):
    wid = lax.axis_index("s") * 2 + lax.axis_index("c")
    base_row = wid * ROWS_PER_W

    def process_chunk(in_v, out_v):
        def row_body(r, _):
            row = in_v.at[r]

            def max_body(i, ms):
                return tuple(
                    jnp.maximum(ms[j], row[pl.ds(i * (U * L) + j * L, L)])
                    for j in range(U)
                )

            ms = lax.fori_loop(
                0, NITER, max_body,
                tuple(jnp.full((L,), -jnp.inf, jnp.float32) for _ in range(U)),
            )
            m = ms[0]
            for j in range(1, U):
                m = jnp.maximum(m, ms[j])
            top = _xlane(m, jnp.maximum)
            thr = top - 1.0

            def app_body(i, vofs):
                vs = [row[pl.ds(i * (U * L) + j * L, L)] for j in range(U)]
                bm = jnp.zeros((L,), jnp.int32)
                for j in range(U):
                    bm = bm | jnp.where(vs[j] > thr, 1 << j, 0)
                b0 = _xlane(bm, jnp.bitwise_or)[0]
                for j in range(U):
                    cbuf[pl.ds(vofs, L)] = vs[j]
                    vofs = vofs + lax.shift_left(
                        lax.shift_right_logical(b0, j) & 1, 4
                    )
                return vofs

            vofs = lax.fori_loop(0, NITER, app_body, jnp.int32(0))

            for j in range(4):
                cbuf[pl.ds(vofs + j * L, L)] = thr
            nquad = lax.shift_right_logical(vofs + 4 * L, 6)

            def scan_sk(tau_c):
                def scan_body(p, accs):
                    accs = list(accs)
                    q = p * (4 * L)
                    for d in range(4):
                        v = cbuf[pl.ds(q + d * L, L)]
                        sel = v > tau_c
                        accs[d] = accs[d] + jnp.where(sel, v, 0.0)
                        accs[4 + d] = accs[4 + d] + jnp.where(sel, 1.0, 0.0)
                    return tuple(accs)

                z = jnp.zeros((L,), jnp.float32)
                r = lax.fori_loop(0, nquad, scan_body, (z,) * 8)
                s_v = (r[0] + r[1]) + (r[2] + r[3])
                k_v = (r[4] + r[5]) + (r[6] + r[7])
                return _xlane(s_v, jnp.add), _xlane(k_v, jnp.add)

            def mic_body(it, carry):
                tau_c, _ = carry
                s_t, k_t = scan_sk(tau_c)
                return ((s_t - 1.0) / k_t, tau_c)

            tau_m, tau_p = lax.fori_loop(0, NM, mic_body, (thr, thr - 1.0))
            taubuf[...] = tau_m

            @pl.when(tau_m[0] != tau_p[0])
            def _():
                def bis_body(b, carry):
                    lo, hi = carry
                    mid = 0.5 * (lo + hi)
                    s_t, k_t = scan_sk(mid)
                    big = (s_t - k_t * mid) > 1.0
                    return jnp.where(big, mid, lo), jnp.where(big, hi, mid)

                lo, hi = lax.fori_loop(0, BITS, bis_body, (tau_m, top))
                taubuf[...] = 0.5 * (lo + hi)

            tau = taubuf[...]

            orow = out_v.at[r]

            def out_body(i, _):
                for j in range(U):
                    off = i * (U * L) + j * L
                    orow[pl.ds(off, L)] = jnp.maximum(row[pl.ds(off, L)] - tau, 0.0)
                return 0

            lax.fori_loop(0, NITER, out_body, 0)
            return 0

        lax.fori_loop(0, R, row_body, 0)

    pltpu.async_copy(x_hbm.at[pl.ds(base_row, R)], in0, si0)

    def super_body(t, _):
        rowa = base_row + (2 * t) * R
        rowb = rowa + R

        pltpu.make_async_copy(x_hbm.at[pl.ds(rowa, R)], in0, si0).wait()
        pltpu.async_copy(x_hbm.at[pl.ds(rowb, R)], in1, si1)

        @pl.when(t > 0)
        def _():
            pltpu.make_async_copy(x_hbm.at[pl.ds(rowa, R)], out0, so0).wait()

        process_chunk(in0, out0)
        pltpu.async_copy(out0, out_hbm.at[pl.ds(rowa, R)], so0)

        pltpu.make_async_copy(x_hbm.at[pl.ds(rowb, R)], in1, si1).wait()

        @pl.when(t < NSUPER - 1)
        def _():
            pltpu.async_copy(x_hbm.at[pl.ds(rowb + R, R)], in0, si0)

        @pl.when(t > 0)
        def _():
            pltpu.make_async_copy(x_hbm.at[pl.ds(rowb, R)], out1, so1).wait()

        process_chunk(in1, out1)
        pltpu.async_copy(out1, out_hbm.at[pl.ds(rowb, R)], so1)
        return 0

    lax.fori_loop(0, NSUPER, super_body, 0)

    pltpu.make_async_copy(x_hbm.at[pl.ds(base_row, R)], out0, so0).wait()
    pltpu.make_async_copy(x_hbm.at[pl.ds(base_row, R)], out1, so1).wait()


def kernel(x):
    out = _sparsemax_sc(x.reshape(NROWS, NCOLS))
    return out.reshape(x.shape)


# speedup vs baseline: 1.1893x; 1.1255x over previous
import functools

import jax
import jax.numpy as jnp
from jax import lax
from jax.experimental import pallas as pl
from jax.experimental.pallas import tpu as pltpu
from jax.experimental.pallas import tpu_sc as plsc

L = 16
NROWS = 8192
NCOLS = 4096
NWORK = 32
ROWS_PER_W = NROWS // NWORK
R = 4
NCHUNK = ROWS_PER_W // R
NSUPER = NCHUNK // 2
CHUNK_BYTES = R * NCOLS * 4
CVEC = NCOLS // L
U = 16
NITER = CVEC // U
NM = 6
BITS = 24

_mesh = plsc.VectorSubcoreMesh(core_axis_name="c", subcore_axis_name="s")

_GDN = lax.GatherDimensionNumbers(
    offset_dims=(), collapsed_slice_dims=(0,), start_index_map=(0,)
)


def _shuf(v, idx):
    return lax.gather(
        v,
        idx[:, None],
        dimension_numbers=_GDN,
        slice_sizes=(1,),
        mode=lax.GatherScatterMode.PROMISE_IN_BOUNDS,
    )


def _xlane(v, op):
    r = v
    for k in (1, 2, 4, 8):
        idx = lax.iota(jnp.int32, L) ^ k
        r = op(r, _shuf(r, idx))
    return r


def _prefix(x):
    lane = lax.iota(jnp.int32, L)
    r = x
    for k in (1, 2, 4, 8):
        sh = _shuf(r, jnp.maximum(lane - k, 0))
        r = r + jnp.where(lane >= k, sh, 0.0)
    return r


@functools.partial(
    pl.kernel,
    mesh=_mesh,
    out_type=jax.ShapeDtypeStruct((NROWS, NCOLS), jnp.float32),
    scratch_types=[
        pltpu.VMEM((R, NCOLS), jnp.float32),
        pltpu.VMEM((R, NCOLS), jnp.float32),
        pltpu.VMEM((R, NCOLS), jnp.float32),
        pltpu.VMEM((R, NCOLS), jnp.float32),
        pltpu.VMEM((NCOLS + 4 * L,), jnp.float32),
        pltpu.VMEM((L,), jnp.float32),
        pltpu.SemaphoreType.DMA,
        pltpu.SemaphoreType.DMA,
        pltpu.SemaphoreType.DMA,
        pltpu.SemaphoreType.DMA,
    ],
)
def _sparsemax_sc(x_hbm, out_hbm, in0, in1, out0, out1, cbuf, taubuf,
                  si0, si1, so0, so1):
    wid = lax.axis_index("s") * 2 + lax.axis_index("c")
    base_row = wid * ROWS_PER_W

    def process_chunk(in_v, out_v):
        def row_body(r, _):
            row = in_v.at[r]

            def max_body(i, ms):
                return tuple(
                    jnp.maximum(ms[j], row[pl.ds(i * (U * L) + j * L, L)])
                    for j in range(U)
                )

            ms = lax.fori_loop(
                0, NITER, max_body,
                tuple(jnp.full((L,), -jnp.inf, jnp.float32) for _ in range(U)),
            )
            m = ms[0]
            for j in range(1, U):
                m = jnp.maximum(m, ms[j])
            top = _xlane(m, jnp.maximum)
            thr = top - 1.0

            thr0 = thr[0]

            def app_body(i, vofs):
                vs = [row[pl.ds(i * (U * L) + j * L, L)] for j in range(U)]
                incs = [
                    jnp.where(_xlane(v, jnp.maximum)[0] > thr0, L, 0) for v in vs
                ]
                for j in range(U):
                    cbuf[pl.ds(vofs, L)] = vs[j]
                    vofs = vofs + incs[j]
                return vofs

            vofs = lax.fori_loop(0, NITER, app_body, jnp.int32(0))

            for j in range(4):
                cbuf[pl.ds(vofs + j * L, L)] = thr
            nquad = lax.shift_right_logical(vofs + 4 * L, 6)

            def scan_sk(tau_c):
                def scan_body(p, accs):
                    s_a, k_a = accs
                    q = p * (4 * L)
                    for d in range(4):
                        v = cbuf[pl.ds(q + d * L, L)]
                        sel = v > tau_c
                        s_a = s_a + jnp.where(sel, v, 0.0)
                        k_a = k_a + jnp.where(sel, 1.0, 0.0)
                    return (s_a, k_a)

                z = jnp.zeros((L,), jnp.float32)
                s_v, k_v = lax.fori_loop(0, nquad, scan_body, (z, z))
                return _xlane(s_v, jnp.add), _xlane(k_v, jnp.add)

            def mic_body(it, carry):
                tau_c, _ = carry
                s_t, k_t = scan_sk(tau_c)
                return ((s_t - 1.0) / k_t, tau_c)

            tau_m, tau_p = lax.fori_loop(0, NM, mic_body, (thr, thr - 1.0))
            taubuf[...] = tau_m

            @pl.when(tau_m[0] != tau_p[0])
            def _():
                def bis_body(b, carry):
                    lo, hi = carry
                    mid = 0.5 * (lo + hi)
                    s_t, k_t = scan_sk(mid)
                    big = (s_t - k_t * mid) > 1.0
                    return jnp.where(big, mid, lo), jnp.where(big, hi, mid)

                lo, hi = lax.fori_loop(0, BITS, bis_body, (tau_m, top))
                taubuf[...] = 0.5 * (lo + hi)

            tau = taubuf[...]

            orow = out_v.at[r]

            def out_body(i, _):
                for j in range(U):
                    off = i * (U * L) + j * L
                    orow[pl.ds(off, L)] = jnp.maximum(row[pl.ds(off, L)] - tau, 0.0)
                return 0

            lax.fori_loop(0, NITER, out_body, 0)
            return 0

        lax.fori_loop(0, R, row_body, 0)

    pltpu.async_copy(x_hbm.at[pl.ds(base_row, R)], in0, si0)

    def super_body(t, _):
        rowa = base_row + (2 * t) * R
        rowb = rowa + R

        pltpu.make_async_copy(x_hbm.at[pl.ds(rowa, R)], in0, si0).wait()
        pltpu.async_copy(x_hbm.at[pl.ds(rowb, R)], in1, si1)

        @pl.when(t > 0)
        def _():
            pltpu.make_async_copy(x_hbm.at[pl.ds(rowa, R)], out0, so0).wait()

        process_chunk(in0, out0)
        pltpu.async_copy(out0, out_hbm.at[pl.ds(rowa, R)], so0)

        pltpu.make_async_copy(x_hbm.at[pl.ds(rowb, R)], in1, si1).wait()

        @pl.when(t < NSUPER - 1)
        def _():
            pltpu.async_copy(x_hbm.at[pl.ds(rowb + R, R)], in0, si0)

        @pl.when(t > 0)
        def _():
            pltpu.make_async_copy(x_hbm.at[pl.ds(rowb, R)], out1, so1).wait()

        process_chunk(in1, out1)
        pltpu.async_copy(out1, out_hbm.at[pl.ds(rowb, R)], so1)
        return 0

    lax.fori_loop(0, NSUPER, super_body, 0)

    pltpu.make_async_copy(x_hbm.at[pl.ds(base_row, R)], out0, so0).wait()
    pltpu.make_async_copy(x_hbm.at[pl.ds(base_row, R)], out1, so1).wait()


def kernel(x):
    out = _sparsemax_sc(x.reshape(NROWS, NCOLS))
    return out.reshape(x.shape)


# speedup vs baseline: 1.2991x; 1.0923x over previous
import functools

import jax
import jax.numpy as jnp
from jax import lax
from jax.experimental import pallas as pl
from jax.experimental.pallas import tpu as pltpu
from jax.experimental.pallas import tpu_sc as plsc

L = 16
NROWS = 8192
NCOLS = 4096
NWORK = 32
ROWS_PER_W = NROWS // NWORK
R = 4
NCHUNK = ROWS_PER_W // R
NSUPER = NCHUNK // 2
CHUNK_BYTES = R * NCOLS * 4
CVEC = NCOLS // L
U = 32
NITER = CVEC // U
NM = 6
BITS = 24

_mesh = plsc.VectorSubcoreMesh(core_axis_name="c", subcore_axis_name="s")

_GDN = lax.GatherDimensionNumbers(
    offset_dims=(), collapsed_slice_dims=(0,), start_index_map=(0,)
)


def _shuf(v, idx):
    return lax.gather(
        v,
        idx[:, None],
        dimension_numbers=_GDN,
        slice_sizes=(1,),
        mode=lax.GatherScatterMode.PROMISE_IN_BOUNDS,
    )


def _xlane(v, op):
    r = v
    for k in (1, 2, 4, 8):
        idx = lax.iota(jnp.int32, L) ^ k
        r = op(r, _shuf(r, idx))
    return r


def _prefix(x):
    lane = lax.iota(jnp.int32, L)
    r = x
    for k in (1, 2, 4, 8):
        sh = _shuf(r, jnp.maximum(lane - k, 0))
        r = r + jnp.where(lane >= k, sh, 0.0)
    return r


@functools.partial(
    pl.kernel,
    mesh=_mesh,
    out_type=jax.ShapeDtypeStruct((NROWS, NCOLS), jnp.float32),
    scratch_types=[
        pltpu.VMEM((R, NCOLS), jnp.float32),
        pltpu.VMEM((R, NCOLS), jnp.float32),
        pltpu.VMEM((R, NCOLS), jnp.float32),
        pltpu.VMEM((R, NCOLS), jnp.float32),
        pltpu.VMEM((NCOLS + 4 * L,), jnp.float32),
        pltpu.VMEM((L,), jnp.float32),
        pltpu.SemaphoreType.DMA,
        pltpu.SemaphoreType.DMA,
        pltpu.SemaphoreType.DMA,
        pltpu.SemaphoreType.DMA,
    ],
)
def _sparsemax_sc(x_hbm, out_hbm, in0, in1, out0, out1, cbuf, taubuf,
                  si0, si1, so0, so1):
    wid = lax.axis_index("s") * 2 + lax.axis_index("c")
    base_row = wid * ROWS_PER_W

    def process_chunk(in_v, out_v):
        def row_body(r, _):
            row = in_v.at[r]

            def max_body(i, ms):
                return tuple(
                    jnp.maximum(ms[j], row[pl.ds(i * (U * L) + j * L, L)])
                    for j in range(U)
                )

            ms = lax.fori_loop(
                0, NITER, max_body,
                tuple(jnp.full((L,), -jnp.inf, jnp.float32) for _ in range(U)),
            )
            m = ms[0]
            for j in range(1, U):
                m = jnp.maximum(m, ms[j])
            top = _xlane(m, jnp.maximum)
            thr = top - 1.0

            thr0 = thr[0]

            def app_body(i, vofs):
                vs = [row[pl.ds(i * (U * L) + j * L, L)] for j in range(U)]
                incs = [
                    jnp.where(_xlane(v, jnp.maximum)[0] > thr0, L, 0) for v in vs
                ]
                for j in range(U):
                    cbuf[pl.ds(vofs, L)] = vs[j]
                    vofs = vofs + incs[j]
                return vofs

            vofs = lax.fori_loop(0, NITER, app_body, jnp.int32(0))

            for j in range(4):
                cbuf[pl.ds(vofs + j * L, L)] = thr
            nquad = lax.shift_right_logical(vofs + 4 * L, 6)

            def scan_sk(tau_c):
                def scan_body(p, accs):
                    s_a, k_a = accs
                    q = p * (4 * L)
                    for d in range(4):
                        v = cbuf[pl.ds(q + d * L, L)]
                        sel = v > tau_c
                        s_a = s_a + jnp.where(sel, v, 0.0)
                        k_a = k_a + jnp.where(sel, 1.0, 0.0)
                    return (s_a, k_a)

                z = jnp.zeros((L,), jnp.float32)
                s_v, k_v = lax.fori_loop(0, nquad, scan_body, (z, z))
                return _xlane(s_v, jnp.add), _xlane(k_v, jnp.add)

            def mic_body(it, carry):
                tau_c, _ = carry
                s_t, k_t = scan_sk(tau_c)
                return ((s_t - 1.0) / k_t, tau_c)

            tau_m, tau_p = lax.fori_loop(0, NM, mic_body, (thr, thr - 1.0))
            taubuf[...] = tau_m

            @pl.when(tau_m[0] != tau_p[0])
            def _():
                def bis_body(b, carry):
                    lo, hi = carry
                    mid = 0.5 * (lo + hi)
                    s_t, k_t = scan_sk(mid)
                    big = (s_t - k_t * mid) > 1.0
                    return jnp.where(big, mid, lo), jnp.where(big, hi, mid)

                lo, hi = lax.fori_loop(0, BITS, bis_body, (tau_m, top))
                taubuf[...] = 0.5 * (lo + hi)

            tau = taubuf[...]

            orow = out_v.at[r]

            def out_body(i, _):
                for j in range(U):
                    off = i * (U * L) + j * L
                    orow[pl.ds(off, L)] = jnp.maximum(row[pl.ds(off, L)] - tau, 0.0)
                return 0

            lax.fori_loop(0, NITER, out_body, 0)
            return 0

        lax.fori_loop(0, R, row_body, 0)

    pltpu.async_copy(x_hbm.at[pl.ds(base_row, R)], in0, si0)

    def super_body(t, _):
        rowa = base_row + (2 * t) * R
        rowb = rowa + R

        pltpu.make_async_copy(x_hbm.at[pl.ds(rowa, R)], in0, si0).wait()
        pltpu.async_copy(x_hbm.at[pl.ds(rowb, R)], in1, si1)

        @pl.when(t > 0)
        def _():
            pltpu.make_async_copy(x_hbm.at[pl.ds(rowa, R)], out0, so0).wait()

        process_chunk(in0, out0)
        pltpu.async_copy(out0, out_hbm.at[pl.ds(rowa, R)], so0)

        pltpu.make_async_copy(x_hbm.at[pl.ds(rowb, R)], in1, si1).wait()

        @pl.when(t < NSUPER - 1)
        def _():
            pltpu.async_copy(x_hbm.at[pl.ds(rowb + R, R)], in0, si0)

        @pl.when(t > 0)
        def _():
            pltpu.make_async_copy(x_hbm.at[pl.ds(rowb, R)], out1, so1).wait()

        process_chunk(in1, out1)
        pltpu.async_copy(out1, out_hbm.at[pl.ds(rowb, R)], so1)
        return 0

    lax.fori_loop(0, NSUPER, super_body, 0)

    pltpu.make_async_copy(x_hbm.at[pl.ds(base_row, R)], out0, so0).wait()
    pltpu.make_async_copy(x_hbm.at[pl.ds(base_row, R)], out1, so1).wait()


def kernel(x):
    out = _sparsemax_sc(x.reshape(NROWS, NCOLS))
    return out.reshape(x.shape)
